# baseline (speedup 1.0000x reference)
import functools

import jax
import jax.numpy as jnp
from jax import lax
from jax.experimental import pallas as pl
from jax.experimental.pallas import tpu as pltpu
from jax.experimental.pallas import tpu_sc as plsc

_NC = 2
_NS = 16
_NW = _NC * _NS

_LANES = 128
_TB = 2048
_CT = 512
_CH = 64
_BIG = 2**30


def _router_body(x_ref, w_ref, b_ref,
                 disp_ref, comb_ref, kept_ref, pmax_ref, counts_ref, psum_ref,
                 xb_ref, *, capacity, trash):
    i = pl.program_id(0)

    @pl.when(i == 0)
    def _init():
        counts_ref[...] = jnp.zeros_like(counts_ref)
        psum_ref[...] = jnp.zeros_like(psum_ref)

    x = x_ref[...]
    xb_ref[...] = x.astype(jnp.bfloat16)
    logits = jnp.dot(x, w_ref[...], preferred_element_type=jnp.float32)
    logits = logits + b_ref[...]
    m = jnp.max(logits, axis=1, keepdims=True)
    ex = jnp.exp(logits - m)
    s = jnp.sum(ex, axis=1, keepdims=True)
    prob = ex / s
    pmax = jnp.max(prob, axis=1, keepdims=True)
    lane = lax.broadcasted_iota(jnp.int32, (_TB, _LANES), 1)
    routes = jnp.min(jnp.where(logits == m, lane, _LANES), axis=1,
                     keepdims=True)
    onehot = (lane == routes).astype(jnp.float32)

    psum_ref[...] += jnp.sum(prob, axis=0, keepdims=True)

    r = lax.broadcasted_iota(jnp.int32, (128, 128), 0)
    c = lax.broadcasted_iota(jnp.int32, (128, 128), 1)
    ltri = (r >= c).astype(jnp.float32)
    run = counts_ref[...]
    parts = []
    for g in range(_TB // 128):
        blk = onehot[g * 128:(g + 1) * 128, :]
        cs = jnp.dot(ltri, blk, preferred_element_type=jnp.float32)
        parts.append(cs + run)
        run = run + cs[127:128, :]
    counts_ref[...] = run
    pos_incl = jnp.concatenate(parts, axis=0)

    posf = jnp.sum(pos_incl * onehot, axis=1, keepdims=True) - 1.0
    pos = posf.astype(jnp.int32)
    keptb = pos < capacity
    clip = jnp.minimum(pos, capacity - 1)
    comb = routes * capacity + clip
    disp_ref[...] = jnp.where(keptb, comb, trash)
    comb_ref[...] = comb
    kept_ref[...] = keptb.astype(jnp.float32)
    pmax_ref[...] = pmax


def _router(xf, w_pad, b_pad, capacity, trash):
    n, d = xf.shape
    nb = n // _TB
    out_shape = [
        jax.ShapeDtypeStruct((n, 1), jnp.int32),
        jax.ShapeDtypeStruct((n, 1), jnp.int32),
        jax.ShapeDtypeStruct((n, 1), jnp.float32),
        jax.ShapeDtypeStruct((n, 1), jnp.float32),
        jax.ShapeDtypeStruct((1, _LANES), jnp.float32),
        jax.ShapeDtypeStruct((1, _LANES), jnp.float32),
        jax.ShapeDtypeStruct((n, d), jnp.bfloat16),
    ]
    tok_spec = pl.BlockSpec((_TB, 1), lambda i: (i, 0))
    acc_spec = pl.BlockSpec((1, _LANES), lambda i: (0, 0))
    row_spec = pl.BlockSpec((_TB, d), lambda i: (i, 0))
    return pl.pallas_call(
        functools.partial(_router_body, capacity=capacity, trash=trash),
        grid=(nb,),
        in_specs=[
            row_spec,
            pl.BlockSpec((d, _LANES), lambda i: (0, 0)),
            pl.BlockSpec((1, _LANES), lambda i: (0, 0)),
        ],
        out_specs=[tok_spec, tok_spec, tok_spec, tok_spec, acc_spec, acc_spec,
                   row_spec],
        out_shape=out_shape,
    )(xf, w_pad, b_pad)


def _sc_dispatch(disp2d, xf, nt):
    n, d = xf.shape
    tpw = n // _NW
    nch = tpw // _CH
    mesh = plsc.VectorSubcoreMesh(core_axis_name="c", subcore_axis_name="s",
                                  num_cores=_NC, num_subcores=_NS)

    dt = xf.dtype

    @functools.partial(
        pl.kernel,
        out_type=jax.ShapeDtypeStruct((nt + 8, d), dt),
        mesh=mesh,
        compiler_params=pltpu.CompilerParams(needs_layout_passes=False),
        scratch_types=[
            pltpu.VMEM((nch, _CH), jnp.int32),
            pltpu.VMEM((2, _CH, d), dt),
            pltpu.SemaphoreType.DMA,
            pltpu.SemaphoreType.DMA,
        ],
    )
    def dispatch(disp_hbm, xf_hbm, buf_hbm, idx_v, rows_v, sem_in, sem_out):
        wid = lax.axis_index("s") * _NC + lax.axis_index("c")
        pltpu.sync_copy(disp_hbm.at[pl.ds(wid * nch, nch)], idx_v)

        def load(k):
            return pltpu.async_copy(
                xf_hbm.at[pl.ds(wid * tpw + k * _CH, _CH)],
                rows_v.at[k % 2], sem_in)

        def scat(k):
            return pltpu.async_copy(
                rows_v.at[k % 2], buf_hbm.at[idx_v.at[k]], sem_out)

        h_out = [None, None]
        h_in = load(0)
        for k in range(nch):
            h_in.wait()
            if k >= 1:
                h_out[(k - 1) % 2].wait()
            if k + 1 < nch:
                h_in = load(k + 1)
            h_out[k % 2] = scat(k)
        h_out[(nch - 1) % 2].wait()

    return dispatch(disp2d, xf)


def _ffn_body(x_ref, w1_ref, b1_ref, w2_ref, b2_ref, o_ref):
    x = x_ref[...]
    h = jnp.dot(x, w1_ref[0], preferred_element_type=jnp.float32)
    h = jnp.maximum(h + b1_ref[0], 0.0)
    o = jnp.dot(h.astype(jnp.bfloat16), w2_ref[0],
                preferred_element_type=jnp.float32)
    o_ref[...] = (o + b2_ref[0]).astype(jnp.bfloat16)


def _ffn(buf_flat, w1, b1, w2, b2, e, cap):
    d = buf_flat.shape[1]
    h = w1.shape[2]
    ct = cap // _CT
    return pl.pallas_call(
        _ffn_body,
        grid=(e, ct),
        in_specs=[
            pl.BlockSpec((_CT, d), lambda i, j: (i * ct + j, 0)),
            pl.BlockSpec((1, d, h), lambda i, j: (i, 0, 0)),
            pl.BlockSpec((1, 1, h), lambda i, j: (i, 0, 0)),
            pl.BlockSpec((1, h, d), lambda i, j: (i, 0, 0)),
            pl.BlockSpec((1, 1, d), lambda i, j: (i, 0, 0)),
        ],
        out_specs=pl.BlockSpec((_CT, d), lambda i, j: (i * ct + j, 0)),
        out_shape=jax.ShapeDtypeStruct((e * cap, d), jnp.bfloat16),
    )(buf_flat, w1, b1.reshape(e, 1, h), w2, b2.reshape(e, 1, d))


def _sc_combine(comb, out_flat):
    n = comb.shape[0]
    d = out_flat.shape[1]
    tpw = n // _NW
    mesh = plsc.VectorSubcoreMesh(core_axis_name="c", subcore_axis_name="s",
                                  num_cores=_NC, num_subcores=_NS)

    dt = out_flat.dtype

    @functools.partial(
        pl.kernel,
        out_type=jax.ShapeDtypeStruct((n, d), dt),
        mesh=mesh,
        compiler_params=pltpu.CompilerParams(needs_layout_passes=False),
        scratch_types=[
            pltpu.VMEM((tpw,), jnp.int32),
            pltpu.VMEM((2, _CH, d), dt),
            pltpu.SemaphoreType.DMA,
            pltpu.SemaphoreType.DMA,
        ],
    )
    def combine(comb_hbm, out_hbm, g_hbm, cidx_v, rows_v, sem_in, sem_out):
        wid = lax.axis_index("s") * _NC + lax.axis_index("c")
        tbase = wid * tpw
        nch = tpw // _CH
        pltpu.sync_copy(comb_hbm.at[pl.ds(tbase, tpw)], cidx_v)

        def load(k):
            return pltpu.async_copy(
                out_hbm.at[cidx_v.at[pl.ds(k * _CH, _CH)]],
                rows_v.at[k % 2], sem_in)

        def store(k):
            return pltpu.async_copy(
                rows_v.at[k % 2], g_hbm.at[pl.ds(tbase + k * _CH, _CH)],
                sem_out)

        h_out = [None, None]
        h_in = load(0)
        for k in range(nch):
            h_in.wait()
            if k >= 1:
                h_out[(k - 1) % 2].wait()
            if k + 1 < nch:
                h_in = load(k + 1)
            h_out[k % 2] = store(k)
        h_out[(nch - 1) % 2].wait()

    return combine(comb, out_flat)


def _combine_body(g_ref, x_ref, k_ref, p_ref, o_ref):
    keep = k_ref[...] > 0.0
    g = g_ref[...].astype(jnp.float32)
    o_ref[...] = jnp.where(keep, g, x_ref[...]) * p_ref[...]


def _combine_tc(gathered, xf, keptf, pmax):
    n, d = xf.shape
    row_spec = pl.BlockSpec((_TB, d), lambda i: (i, 0))
    col_spec = pl.BlockSpec((_TB, 1), lambda i: (i, 0))
    return pl.pallas_call(
        _combine_body,
        grid=(n // _TB,),
        in_specs=[row_spec, row_spec, col_spec, col_spec],
        out_specs=row_spec,
        out_shape=jax.ShapeDtypeStruct((n, d), jnp.float32),
    )(gathered, xf, keptf, pmax)


def kernel(x, W_switch, b_switch, W1, b1, W2, b2):
    b, s, d = x.shape
    e = W_switch.shape[1]
    n = b * s
    capacity = int(1.25 * n / e)
    nt = e * capacity
    xf = x.reshape(n, d)

    w_pad = jnp.zeros((d, _LANES), jnp.float32).at[:, :e].set(W_switch)
    b_pad = jnp.full((1, _LANES), -1e30, jnp.float32).at[0, :e].set(b_switch)

    (disp, comb, keptf, pmax, counts_l, psum_l,
     xb) = _router(xf, w_pad, b_pad, capacity, nt)

    xb_i = lax.bitcast_convert_type(xb.reshape(n, d // 2, 2), jnp.int32)
    buf_i = _sc_dispatch(disp.reshape(n // _CH, _CH), xb_i, nt)
    buf_b = lax.bitcast_convert_type(buf_i, jnp.bfloat16).reshape(nt + 8, d)
    out_flat = _ffn(buf_b, W1.astype(jnp.bfloat16), b1,
                    W2.astype(jnp.bfloat16), b2, e, capacity)
    out_i = lax.bitcast_convert_type(out_flat.reshape(nt, d // 2, 2),
                                     jnp.int32)
    g_i = _sc_combine(comb.reshape(n), out_i)
    gathered = lax.bitcast_convert_type(g_i, jnp.bfloat16).reshape(n, d)
    final = _combine_tc(gathered, xf, keptf, pmax)

    counts = counts_l[0, :e]
    psum = psum_l[0, :e]
    n_dropped = jnp.sum(jnp.maximum(counts - capacity, 0.0)).astype(jnp.int32)
    return (final.reshape(b, s, d), counts, psum, n_dropped,
            pmax.reshape(n))


# speedup vs baseline: 4.7828x; 4.7828x over previous
import functools

import jax
import jax.numpy as jnp
from jax import lax
from jax.experimental import pallas as pl
from jax.experimental.pallas import tpu as pltpu
from jax.experimental.pallas import tpu_sc as plsc

_NC = 2
_NS = 16
_NW = _NC * _NS

_LANES = 128
_TB = 2048
_CT = 512
_CH = 64
_BIG = 2**30


def _router_body(x_ref, w_ref, b_ref,
                 disp_ref, comb_ref, kept_ref, pmax_ref, counts_ref, psum_ref,
                 *, capacity, trash):
    i = pl.program_id(0)

    @pl.when(i == 0)
    def _init():
        counts_ref[...] = jnp.zeros_like(counts_ref)
        psum_ref[...] = jnp.zeros_like(psum_ref)

    x = x_ref[...]
    logits = jnp.dot(x, w_ref[...], preferred_element_type=jnp.float32)
    logits = logits + b_ref[...]
    m = jnp.max(logits, axis=1, keepdims=True)
    ex = jnp.exp(logits - m)
    s = jnp.sum(ex, axis=1, keepdims=True)
    prob = ex / s
    pmax = jnp.max(prob, axis=1, keepdims=True)
    lane = lax.broadcasted_iota(jnp.int32, (_TB, _LANES), 1)
    routes = jnp.min(jnp.where(logits == m, lane, _LANES), axis=1,
                     keepdims=True)
    onehot = (lane == routes).astype(jnp.float32)

    psum_ref[...] += jnp.sum(prob, axis=0, keepdims=True)

    r = lax.broadcasted_iota(jnp.int32, (128, 128), 0)
    c = lax.broadcasted_iota(jnp.int32, (128, 128), 1)
    ltri = (r >= c).astype(jnp.float32)
    run = counts_ref[...]
    parts = []
    for g in range(_TB // 128):
        blk = onehot[g * 128:(g + 1) * 128, :]
        cs = jnp.dot(ltri, blk, preferred_element_type=jnp.float32)
        parts.append(cs + run)
        run = run + cs[127:128, :]
    counts_ref[...] = run
    pos_incl = jnp.concatenate(parts, axis=0)

    posf = jnp.sum(pos_incl * onehot, axis=1, keepdims=True) - 1.0
    pos = posf.astype(jnp.int32)
    keptb = pos < capacity
    clip = jnp.minimum(pos, capacity - 1)
    comb = routes * capacity + clip
    disp_ref[...] = jnp.where(keptb, comb, trash)
    comb_ref[...] = comb
    kept_ref[...] = keptb.astype(jnp.float32)
    pmax_ref[...] = pmax


def _router(xf, w_pad, b_pad, capacity, trash):
    n, d = xf.shape
    nb = n // _TB
    out_shape = [
        jax.ShapeDtypeStruct((n, 1), jnp.int32),
        jax.ShapeDtypeStruct((n, 1), jnp.int32),
        jax.ShapeDtypeStruct((n, 1), jnp.float32),
        jax.ShapeDtypeStruct((n, 1), jnp.float32),
        jax.ShapeDtypeStruct((1, _LANES), jnp.float32),
        jax.ShapeDtypeStruct((1, _LANES), jnp.float32),
    ]
    tok_spec = pl.BlockSpec((_TB, 1), lambda i: (i, 0))
    acc_spec = pl.BlockSpec((1, _LANES), lambda i: (0, 0))
    row_spec = pl.BlockSpec((_TB, d), lambda i: (i, 0))
    return pl.pallas_call(
        functools.partial(_router_body, capacity=capacity, trash=trash),
        grid=(nb,),
        in_specs=[
            row_spec,
            pl.BlockSpec((d, _LANES), lambda i: (0, 0)),
            pl.BlockSpec((1, _LANES), lambda i: (0, 0)),
        ],
        out_specs=[tok_spec, tok_spec, tok_spec, tok_spec, acc_spec, acc_spec],
        out_shape=out_shape,
    )(xf, w_pad, b_pad)


def _sc_dispatch(disp2d, xf, nt):
    n, d = xf.shape
    tpw = n // _NW
    nch = tpw // _CH
    mesh = plsc.VectorSubcoreMesh(core_axis_name="c", subcore_axis_name="s",
                                  num_cores=_NC, num_subcores=_NS)

    dt = xf.dtype

    @functools.partial(
        pl.kernel,
        out_type=jax.ShapeDtypeStruct((nt + 8, d), dt),
        mesh=mesh,
        compiler_params=pltpu.CompilerParams(needs_layout_passes=False),
        scratch_types=[
            pltpu.VMEM((nch, _CH), jnp.int32),
            pltpu.VMEM((2, _CH, d), dt),
            pltpu.SemaphoreType.DMA,
            pltpu.SemaphoreType.DMA,
        ],
    )
    def dispatch(disp_hbm, xf_hbm, buf_hbm, idx_v, rows_v, sem_in, sem_out):
        wid = lax.axis_index("s") * _NC + lax.axis_index("c")
        pltpu.sync_copy(disp_hbm.at[pl.ds(wid * nch, nch)], idx_v)

        def load(k):
            return pltpu.async_copy(
                xf_hbm.at[pl.ds(wid * tpw + k * _CH, _CH)],
                rows_v.at[k % 2], sem_in)

        def scat(k):
            return pltpu.async_copy(
                rows_v.at[k % 2], buf_hbm.at[idx_v.at[k]], sem_out)

        h_out = [None, None]
        h_in = load(0)
        for k in range(nch):
            h_in.wait()
            if k >= 1:
                h_out[(k - 1) % 2].wait()
            if k + 1 < nch:
                h_in = load(k + 1)
            h_out[k % 2] = scat(k)
        h_out[(nch - 1) % 2].wait()

    return dispatch(disp2d, xf)


def _ffn_body(x_ref, w1_ref, b1_ref, w2_ref, b2_ref, o_ref):
    x = x_ref[...].astype(jnp.bfloat16)
    h = jnp.dot(x, w1_ref[0], preferred_element_type=jnp.float32)
    h = jnp.maximum(h + b1_ref[0], 0.0)
    o = jnp.dot(h.astype(jnp.bfloat16), w2_ref[0],
                preferred_element_type=jnp.float32)
    o_ref[...] = o + b2_ref[0]


def _ffn(buf_flat, w1, b1, w2, b2, e, cap):
    d = buf_flat.shape[1]
    h = w1.shape[2]
    ct = cap // _CT
    return pl.pallas_call(
        _ffn_body,
        grid=(e, ct),
        in_specs=[
            pl.BlockSpec((_CT, d), lambda i, j: (i * ct + j, 0)),
            pl.BlockSpec((1, d, h), lambda i, j: (i, 0, 0)),
            pl.BlockSpec((1, 1, h), lambda i, j: (i, 0, 0)),
            pl.BlockSpec((1, h, d), lambda i, j: (i, 0, 0)),
            pl.BlockSpec((1, 1, d), lambda i, j: (i, 0, 0)),
        ],
        out_specs=pl.BlockSpec((_CT, d), lambda i, j: (i * ct + j, 0)),
        out_shape=jax.ShapeDtypeStruct((e * cap, d), jnp.float32),
    )(buf_flat, w1, b1.reshape(e, 1, h), w2, b2.reshape(e, 1, d))


def _sc_combine(comb, out_flat):
    n = comb.shape[0]
    d = out_flat.shape[1]
    tpw = n // _NW
    mesh = plsc.VectorSubcoreMesh(core_axis_name="c", subcore_axis_name="s",
                                  num_cores=_NC, num_subcores=_NS)

    dt = out_flat.dtype

    @functools.partial(
        pl.kernel,
        out_type=jax.ShapeDtypeStruct((n, d), dt),
        mesh=mesh,
        compiler_params=pltpu.CompilerParams(needs_layout_passes=False),
        scratch_types=[
            pltpu.VMEM((tpw,), jnp.int32),
            pltpu.VMEM((2, _CH, d), dt),
            pltpu.SemaphoreType.DMA,
            pltpu.SemaphoreType.DMA,
        ],
    )
    def combine(comb_hbm, out_hbm, g_hbm, cidx_v, rows_v, sem_in, sem_out):
        wid = lax.axis_index("s") * _NC + lax.axis_index("c")
        tbase = wid * tpw
        nch = tpw // _CH
        pltpu.sync_copy(comb_hbm.at[pl.ds(tbase, tpw)], cidx_v)

        def load(k):
            return pltpu.async_copy(
                out_hbm.at[cidx_v.at[pl.ds(k * _CH, _CH)]],
                rows_v.at[k % 2], sem_in)

        def store(k):
            return pltpu.async_copy(
                rows_v.at[k % 2], g_hbm.at[pl.ds(tbase + k * _CH, _CH)],
                sem_out)

        h_out = [None, None]
        h_in = load(0)
        for k in range(nch):
            h_in.wait()
            if k >= 1:
                h_out[(k - 1) % 2].wait()
            if k + 1 < nch:
                h_in = load(k + 1)
            h_out[k % 2] = store(k)
        h_out[(nch - 1) % 2].wait()

    return combine(comb, out_flat)


def _combine_body(g_ref, x_ref, k_ref, p_ref, o_ref):
    keep = k_ref[...] > 0.0
    o_ref[...] = jnp.where(keep, g_ref[...], x_ref[...]) * p_ref[...]


def _combine_tc(gathered, xf, keptf, pmax):
    n, d = xf.shape
    row_spec = pl.BlockSpec((_TB, d), lambda i: (i, 0))
    col_spec = pl.BlockSpec((_TB, 1), lambda i: (i, 0))
    return pl.pallas_call(
        _combine_body,
        grid=(n // _TB,),
        in_specs=[row_spec, row_spec, col_spec, col_spec],
        out_specs=row_spec,
        out_shape=jax.ShapeDtypeStruct((n, d), jnp.float32),
    )(gathered, xf, keptf, pmax)


def kernel(x, W_switch, b_switch, W1, b1, W2, b2):
    b, s, d = x.shape
    e = W_switch.shape[1]
    n = b * s
    capacity = int(1.25 * n / e)
    nt = e * capacity
    xf = x.reshape(n, d)

    w_pad = jnp.zeros((d, _LANES), jnp.float32).at[:, :e].set(W_switch)
    b_pad = jnp.full((1, _LANES), -1e30, jnp.float32).at[0, :e].set(b_switch)

    disp, comb, keptf, pmax, counts_l, psum_l = _router(xf, w_pad, b_pad,
                                                        capacity, nt)

    buf = _sc_dispatch(disp.reshape(n // _CH, _CH), xf, nt)
    out_flat = _ffn(buf, W1.astype(jnp.bfloat16), b1,
                    W2.astype(jnp.bfloat16), b2, e, capacity)
    gathered = _sc_combine(comb.reshape(n), out_flat)
    final = _combine_tc(gathered, xf, keptf, pmax)

    counts = counts_l[0, :e]
    psum = psum_l[0, :e]
    n_dropped = jnp.sum(jnp.maximum(counts - capacity, 0.0)).astype(jnp.int32)
    return (final.reshape(b, s, d), counts, psum, n_dropped,
            pmax.reshape(n))


# speedup vs baseline: 4.9584x; 1.0367x over previous
import functools

import jax
import jax.numpy as jnp
from jax import lax
from jax.experimental import pallas as pl
from jax.experimental.pallas import tpu as pltpu
from jax.experimental.pallas import tpu_sc as plsc

_NC = 2
_NS = 16
_NW = _NC * _NS

_LANES = 128
_TB = 2048
_CT = 1024
_CH = 64
_BIG = 2**30


def _router_body(x_ref, w_ref, b_ref,
                 disp_ref, comb_ref, kept_ref, pmax_ref, counts_ref, psum_ref,
                 *, capacity, trash):
    i = pl.program_id(0)

    @pl.when(i == 0)
    def _init():
        counts_ref[...] = jnp.zeros_like(counts_ref)
        psum_ref[...] = jnp.zeros_like(psum_ref)

    x = x_ref[...]
    logits = jnp.dot(x, w_ref[...], preferred_element_type=jnp.float32)
    logits = logits + b_ref[...]
    m = jnp.max(logits, axis=1, keepdims=True)
    ex = jnp.exp(logits - m)
    s = jnp.sum(ex, axis=1, keepdims=True)
    prob = ex / s
    pmax = jnp.max(prob, axis=1, keepdims=True)
    lane = lax.broadcasted_iota(jnp.int32, (_TB, _LANES), 1)
    routes = jnp.min(jnp.where(logits == m, lane, _LANES), axis=1,
                     keepdims=True)
    onehot = (lane == routes).astype(jnp.float32)

    psum_ref[...] += jnp.sum(prob, axis=0, keepdims=True)

    r = lax.broadcasted_iota(jnp.int32, (128, 128), 0)
    c = lax.broadcasted_iota(jnp.int32, (128, 128), 1)
    ltri = (r >= c).astype(jnp.float32)
    run = counts_ref[...]
    parts = []
    for g in range(_TB // 128):
        blk = onehot[g * 128:(g + 1) * 128, :]
        cs = jnp.dot(ltri, blk, preferred_element_type=jnp.float32)
        parts.append(cs + run)
        run = run + cs[127:128, :]
    counts_ref[...] = run
    pos_incl = jnp.concatenate(parts, axis=0)

    posf = jnp.sum(pos_incl * onehot, axis=1, keepdims=True) - 1.0
    pos = posf.astype(jnp.int32)
    keptb = pos < capacity
    clip = jnp.minimum(pos, capacity - 1)
    comb = routes * capacity + clip
    disp_ref[...] = jnp.where(keptb, comb, trash)
    comb_ref[...] = comb
    kept_ref[...] = keptb.astype(jnp.float32)
    pmax_ref[...] = pmax


def _router(xf, w_pad, b_pad, capacity, trash):
    n, d = xf.shape
    nb = n // _TB
    out_shape = [
        jax.ShapeDtypeStruct((n, 1), jnp.int32),
        jax.ShapeDtypeStruct((n, 1), jnp.int32),
        jax.ShapeDtypeStruct((n, 1), jnp.float32),
        jax.ShapeDtypeStruct((n, 1), jnp.float32),
        jax.ShapeDtypeStruct((1, _LANES), jnp.float32),
        jax.ShapeDtypeStruct((1, _LANES), jnp.float32),
    ]
    tok_spec = pl.BlockSpec((_TB, 1), lambda i: (i, 0))
    acc_spec = pl.BlockSpec((1, _LANES), lambda i: (0, 0))
    row_spec = pl.BlockSpec((_TB, d), lambda i: (i, 0))
    return pl.pallas_call(
        functools.partial(_router_body, capacity=capacity, trash=trash),
        grid=(nb,),
        in_specs=[
            row_spec,
            pl.BlockSpec((d, _LANES), lambda i: (0, 0)),
            pl.BlockSpec((1, _LANES), lambda i: (0, 0)),
        ],
        out_specs=[tok_spec, tok_spec, tok_spec, tok_spec, acc_spec, acc_spec],
        out_shape=out_shape,
    )(xf, w_pad, b_pad)


def _sc_dispatch(disp2d, xf, nt):
    n, d = xf.shape
    tpw = n // _NW
    nch = tpw // _CH
    mesh = plsc.VectorSubcoreMesh(core_axis_name="c", subcore_axis_name="s",
                                  num_cores=_NC, num_subcores=_NS)

    dt = xf.dtype

    @functools.partial(
        pl.kernel,
        out_type=jax.ShapeDtypeStruct((nt + 8, d), dt),
        mesh=mesh,
        compiler_params=pltpu.CompilerParams(needs_layout_passes=False),
        scratch_types=[
            pltpu.VMEM((nch, _CH), jnp.int32),
            pltpu.VMEM((2, _CH, d), dt),
            pltpu.SemaphoreType.DMA,
            pltpu.SemaphoreType.DMA,
        ],
    )
    def dispatch(disp_hbm, xf_hbm, buf_hbm, idx_v, rows_v, sem_in, sem_out):
        wid = lax.axis_index("s") * _NC + lax.axis_index("c")
        pltpu.sync_copy(disp_hbm.at[pl.ds(wid * nch, nch)], idx_v)

        def load(k):
            return pltpu.async_copy(
                xf_hbm.at[pl.ds(wid * tpw + k * _CH, _CH)],
                rows_v.at[k % 2], sem_in)

        def scat(k):
            return pltpu.async_copy(
                rows_v.at[k % 2], buf_hbm.at[idx_v.at[k]], sem_out)

        h_out = [None, None]
        h_in = load(0)
        for k in range(nch):
            h_in.wait()
            if k >= 1:
                h_out[(k - 1) % 2].wait()
            if k + 1 < nch:
                h_in = load(k + 1)
            h_out[k % 2] = scat(k)
        h_out[(nch - 1) % 2].wait()

    return dispatch(disp2d, xf)


def _ffn_body(x_ref, w1_ref, b1_ref, w2_ref, b2_ref, o_ref):
    x = x_ref[...].astype(jnp.bfloat16)
    h = jnp.dot(x, w1_ref[0], preferred_element_type=jnp.float32)
    h = jnp.maximum(h + b1_ref[0], 0.0)
    o = jnp.dot(h.astype(jnp.bfloat16), w2_ref[0],
                preferred_element_type=jnp.float32)
    o_ref[...] = o + b2_ref[0]


def _ffn(buf_flat, w1, b1, w2, b2, e, cap):
    d = buf_flat.shape[1]
    h = w1.shape[2]
    ct = cap // _CT
    return pl.pallas_call(
        _ffn_body,
        grid=(e, ct),
        in_specs=[
            pl.BlockSpec((_CT, d), lambda i, j: (i * ct + j, 0)),
            pl.BlockSpec((1, d, h), lambda i, j: (i, 0, 0)),
            pl.BlockSpec((1, 1, h), lambda i, j: (i, 0, 0)),
            pl.BlockSpec((1, h, d), lambda i, j: (i, 0, 0)),
            pl.BlockSpec((1, 1, d), lambda i, j: (i, 0, 0)),
        ],
        out_specs=pl.BlockSpec((_CT, d), lambda i, j: (i * ct + j, 0)),
        out_shape=jax.ShapeDtypeStruct((e * cap, d), jnp.float32),
    )(buf_flat, w1, b1.reshape(e, 1, h), w2, b2.reshape(e, 1, d))


def _sc_combine(comb, out_flat):
    n = comb.shape[0]
    d = out_flat.shape[1]
    tpw = n // _NW
    mesh = plsc.VectorSubcoreMesh(core_axis_name="c", subcore_axis_name="s",
                                  num_cores=_NC, num_subcores=_NS)

    dt = out_flat.dtype

    @functools.partial(
        pl.kernel,
        out_type=jax.ShapeDtypeStruct((n, d), dt),
        mesh=mesh,
        compiler_params=pltpu.CompilerParams(needs_layout_passes=False),
        scratch_types=[
            pltpu.VMEM((tpw,), jnp.int32),
            pltpu.VMEM((2, _CH, d), dt),
            pltpu.SemaphoreType.DMA,
            pltpu.SemaphoreType.DMA,
        ],
    )
    def combine(comb_hbm, out_hbm, g_hbm, cidx_v, rows_v, sem_in, sem_out):
        wid = lax.axis_index("s") * _NC + lax.axis_index("c")
        tbase = wid * tpw
        nch = tpw // _CH
        pltpu.sync_copy(comb_hbm.at[pl.ds(tbase, tpw)], cidx_v)

        def load(k):
            return pltpu.async_copy(
                out_hbm.at[cidx_v.at[pl.ds(k * _CH, _CH)]],
                rows_v.at[k % 2], sem_in)

        def store(k):
            return pltpu.async_copy(
                rows_v.at[k % 2], g_hbm.at[pl.ds(tbase + k * _CH, _CH)],
                sem_out)

        h_out = [None, None]
        h_in = load(0)
        for k in range(nch):
            h_in.wait()
            if k >= 1:
                h_out[(k - 1) % 2].wait()
            if k + 1 < nch:
                h_in = load(k + 1)
            h_out[k % 2] = store(k)
        h_out[(nch - 1) % 2].wait()

    return combine(comb, out_flat)


def _combine_body(g_ref, x_ref, k_ref, p_ref, o_ref):
    keep = k_ref[...] > 0.0
    o_ref[...] = jnp.where(keep, g_ref[...], x_ref[...]) * p_ref[...]


def _combine_tc(gathered, xf, keptf, pmax):
    n, d = xf.shape
    row_spec = pl.BlockSpec((_TB, d), lambda i: (i, 0))
    col_spec = pl.BlockSpec((_TB, 1), lambda i: (i, 0))
    return pl.pallas_call(
        _combine_body,
        grid=(n // _TB,),
        in_specs=[row_spec, row_spec, col_spec, col_spec],
        out_specs=row_spec,
        out_shape=jax.ShapeDtypeStruct((n, d), jnp.float32),
    )(gathered, xf, keptf, pmax)


def kernel(x, W_switch, b_switch, W1, b1, W2, b2):
    b, s, d = x.shape
    e = W_switch.shape[1]
    n = b * s
    capacity = int(1.25 * n / e)
    nt = e * capacity
    xf = x.reshape(n, d)

    w_pad = jnp.zeros((d, _LANES), jnp.float32).at[:, :e].set(W_switch)
    b_pad = jnp.full((1, _LANES), -1e30, jnp.float32).at[0, :e].set(b_switch)

    disp, comb, keptf, pmax, counts_l, psum_l = _router(xf, w_pad, b_pad,
                                                        capacity, nt)

    buf = _sc_dispatch(disp.reshape(n // _CH, _CH), xf, nt)
    out_flat = _ffn(buf, W1.astype(jnp.bfloat16), b1,
                    W2.astype(jnp.bfloat16), b2, e, capacity)
    gathered = _sc_combine(comb.reshape(n), out_flat)
    final = _combine_tc(gathered, xf, keptf, pmax)

    counts = counts_l[0, :e]
    psum = psum_l[0, :e]
    n_dropped = jnp.sum(jnp.maximum(counts - capacity, 0.0)).astype(jnp.int32)
    return (final.reshape(b, s, d), counts, psum, n_dropped,
            pmax.reshape(n))


# speedup vs baseline: 5.0493x; 1.0183x over previous
import functools

import jax
import jax.numpy as jnp
from jax import lax
from jax.experimental import pallas as pl
from jax.experimental.pallas import tpu as pltpu
from jax.experimental.pallas import tpu_sc as plsc

_NC = 2
_NS = 16
_NW = _NC * _NS

_LANES = 128
_TB = 2048
_CT = 2560
_CH = 64
_BIG = 2**30


def _router_body(x_ref, w_ref, b_ref,
                 disp_ref, comb_ref, kept_ref, pmax_ref, counts_ref, psum_ref,
                 *, capacity, trash):
    i = pl.program_id(0)

    @pl.when(i == 0)
    def _init():
        counts_ref[...] = jnp.zeros_like(counts_ref)
        psum_ref[...] = jnp.zeros_like(psum_ref)

    x = x_ref[...]
    logits = jnp.dot(x, w_ref[...], preferred_element_type=jnp.float32)
    logits = logits + b_ref[...]
    m = jnp.max(logits, axis=1, keepdims=True)
    ex = jnp.exp(logits - m)
    s = jnp.sum(ex, axis=1, keepdims=True)
    prob = ex / s
    pmax = jnp.max(prob, axis=1, keepdims=True)
    lane = lax.broadcasted_iota(jnp.int32, (_TB, _LANES), 1)
    routes = jnp.min(jnp.where(logits == m, lane, _LANES), axis=1,
                     keepdims=True)
    onehot = (lane == routes).astype(jnp.float32)

    psum_ref[...] += jnp.sum(prob, axis=0, keepdims=True)

    r = lax.broadcasted_iota(jnp.int32, (128, 128), 0)
    c = lax.broadcasted_iota(jnp.int32, (128, 128), 1)
    ltri = (r >= c).astype(jnp.float32)
    run = counts_ref[...]
    parts = []
    for g in range(_TB // 128):
        blk = onehot[g * 128:(g + 1) * 128, :]
        cs = jnp.dot(ltri, blk, preferred_element_type=jnp.float32)
        parts.append(cs + run)
        run = run + cs[127:128, :]
    counts_ref[...] = run
    pos_incl = jnp.concatenate(parts, axis=0)

    posf = jnp.sum(pos_incl * onehot, axis=1, keepdims=True) - 1.0
    pos = posf.astype(jnp.int32)
    keptb = pos < capacity
    clip = jnp.minimum(pos, capacity - 1)
    comb = routes * capacity + clip
    disp_ref[...] = jnp.where(keptb, comb, trash)
    comb_ref[...] = comb
    kept_ref[...] = keptb.astype(jnp.float32)
    pmax_ref[...] = pmax


def _router(xf, w_pad, b_pad, capacity, trash):
    n, d = xf.shape
    nb = n // _TB
    out_shape = [
        jax.ShapeDtypeStruct((n, 1), jnp.int32),
        jax.ShapeDtypeStruct((n, 1), jnp.int32),
        jax.ShapeDtypeStruct((n, 1), jnp.float32),
        jax.ShapeDtypeStruct((n, 1), jnp.float32),
        jax.ShapeDtypeStruct((1, _LANES), jnp.float32),
        jax.ShapeDtypeStruct((1, _LANES), jnp.float32),
    ]
    tok_spec = pl.BlockSpec((_TB, 1), lambda i: (i, 0))
    acc_spec = pl.BlockSpec((1, _LANES), lambda i: (0, 0))
    row_spec = pl.BlockSpec((_TB, d), lambda i: (i, 0))
    return pl.pallas_call(
        functools.partial(_router_body, capacity=capacity, trash=trash),
        grid=(nb,),
        in_specs=[
            row_spec,
            pl.BlockSpec((d, _LANES), lambda i: (0, 0)),
            pl.BlockSpec((1, _LANES), lambda i: (0, 0)),
        ],
        out_specs=[tok_spec, tok_spec, tok_spec, tok_spec, acc_spec, acc_spec],
        out_shape=out_shape,
    )(xf, w_pad, b_pad)


def _sc_dispatch(disp2d, xf, nt):
    n, d = xf.shape
    tpw = n // _NW
    nch = tpw // _CH
    mesh = plsc.VectorSubcoreMesh(core_axis_name="c", subcore_axis_name="s",
                                  num_cores=_NC, num_subcores=_NS)

    dt = xf.dtype

    @functools.partial(
        pl.kernel,
        out_type=jax.ShapeDtypeStruct((nt + 8, d), dt),
        mesh=mesh,
        compiler_params=pltpu.CompilerParams(needs_layout_passes=False),
        scratch_types=[
            pltpu.VMEM((nch, _CH), jnp.int32),
            pltpu.VMEM((2, _CH, d), dt),
            pltpu.SemaphoreType.DMA,
            pltpu.SemaphoreType.DMA,
        ],
    )
    def dispatch(disp_hbm, xf_hbm, buf_hbm, idx_v, rows_v, sem_in, sem_out):
        wid = lax.axis_index("s") * _NC + lax.axis_index("c")
        pltpu.sync_copy(disp_hbm.at[pl.ds(wid * nch, nch)], idx_v)

        def load(k):
            return pltpu.async_copy(
                xf_hbm.at[pl.ds(wid * tpw + k * _CH, _CH)],
                rows_v.at[k % 2], sem_in)

        def scat(k):
            return pltpu.async_copy(
                rows_v.at[k % 2], buf_hbm.at[idx_v.at[k]], sem_out)

        h_out = [None, None]
        h_in = load(0)
        for k in range(nch):
            h_in.wait()
            if k >= 1:
                h_out[(k - 1) % 2].wait()
            if k + 1 < nch:
                h_in = load(k + 1)
            h_out[k % 2] = scat(k)
        h_out[(nch - 1) % 2].wait()

    return dispatch(disp2d, xf)


def _ffn_body(x_ref, w1_ref, b1_ref, w2_ref, b2_ref, o_ref):
    x = x_ref[...].astype(jnp.bfloat16)
    h = jnp.dot(x, w1_ref[0], preferred_element_type=jnp.float32)
    h = jnp.maximum(h + b1_ref[0], 0.0)
    o = jnp.dot(h.astype(jnp.bfloat16), w2_ref[0],
                preferred_element_type=jnp.float32)
    o_ref[...] = o + b2_ref[0]


def _ffn(buf_flat, w1, b1, w2, b2, e, cap):
    d = buf_flat.shape[1]
    h = w1.shape[2]
    ct = cap // _CT
    return pl.pallas_call(
        _ffn_body,
        grid=(e, ct),
        in_specs=[
            pl.BlockSpec((_CT, d), lambda i, j: (i * ct + j, 0)),
            pl.BlockSpec((1, d, h), lambda i, j: (i, 0, 0)),
            pl.BlockSpec((1, 1, h), lambda i, j: (i, 0, 0)),
            pl.BlockSpec((1, h, d), lambda i, j: (i, 0, 0)),
            pl.BlockSpec((1, 1, d), lambda i, j: (i, 0, 0)),
        ],
        out_specs=pl.BlockSpec((_CT, d), lambda i, j: (i * ct + j, 0)),
        out_shape=jax.ShapeDtypeStruct((e * cap, d), jnp.float32),
    )(buf_flat, w1, b1.reshape(e, 1, h), w2, b2.reshape(e, 1, d))


def _sc_combine(comb, out_flat):
    n = comb.shape[0]
    d = out_flat.shape[1]
    tpw = n // _NW
    mesh = plsc.VectorSubcoreMesh(core_axis_name="c", subcore_axis_name="s",
                                  num_cores=_NC, num_subcores=_NS)

    dt = out_flat.dtype

    @functools.partial(
        pl.kernel,
        out_type=jax.ShapeDtypeStruct((n, d), dt),
        mesh=mesh,
        compiler_params=pltpu.CompilerParams(needs_layout_passes=False),
        scratch_types=[
            pltpu.VMEM((tpw,), jnp.int32),
            pltpu.VMEM((2, _CH, d), dt),
            pltpu.SemaphoreType.DMA,
            pltpu.SemaphoreType.DMA,
        ],
    )
    def combine(comb_hbm, out_hbm, g_hbm, cidx_v, rows_v, sem_in, sem_out):
        wid = lax.axis_index("s") * _NC + lax.axis_index("c")
        tbase = wid * tpw
        nch = tpw // _CH
        pltpu.sync_copy(comb_hbm.at[pl.ds(tbase, tpw)], cidx_v)

        def load(k):
            return pltpu.async_copy(
                out_hbm.at[cidx_v.at[pl.ds(k * _CH, _CH)]],
                rows_v.at[k % 2], sem_in)

        def store(k):
            return pltpu.async_copy(
                rows_v.at[k % 2], g_hbm.at[pl.ds(tbase + k * _CH, _CH)],
                sem_out)

        h_out = [None, None]
        h_in = load(0)
        for k in range(nch):
            h_in.wait()
            if k >= 1:
                h_out[(k - 1) % 2].wait()
            if k + 1 < nch:
                h_in = load(k + 1)
            h_out[k % 2] = store(k)
        h_out[(nch - 1) % 2].wait()

    return combine(comb, out_flat)


def _combine_body(g_ref, x_ref, k_ref, p_ref, o_ref):
    keep = k_ref[...] > 0.0
    o_ref[...] = jnp.where(keep, g_ref[...], x_ref[...]) * p_ref[...]


def _combine_tc(gathered, xf, keptf, pmax):
    n, d = xf.shape
    row_spec = pl.BlockSpec((_TB, d), lambda i: (i, 0))
    col_spec = pl.BlockSpec((_TB, 1), lambda i: (i, 0))
    return pl.pallas_call(
        _combine_body,
        grid=(n // _TB,),
        in_specs=[row_spec, row_spec, col_spec, col_spec],
        out_specs=row_spec,
        out_shape=jax.ShapeDtypeStruct((n, d), jnp.float32),
    )(gathered, xf, keptf, pmax)


def kernel(x, W_switch, b_switch, W1, b1, W2, b2):
    b, s, d = x.shape
    e = W_switch.shape[1]
    n = b * s
    capacity = int(1.25 * n / e)
    nt = e * capacity
    xf = x.reshape(n, d)

    w_pad = jnp.zeros((d, _LANES), jnp.float32).at[:, :e].set(W_switch)
    b_pad = jnp.full((1, _LANES), -1e30, jnp.float32).at[0, :e].set(b_switch)

    disp, comb, keptf, pmax, counts_l, psum_l = _router(xf, w_pad, b_pad,
                                                        capacity, nt)

    buf = _sc_dispatch(disp.reshape(n // _CH, _CH), xf, nt)
    out_flat = _ffn(buf, W1.astype(jnp.bfloat16), b1,
                    W2.astype(jnp.bfloat16), b2, e, capacity)
    gathered = _sc_combine(comb.reshape(n), out_flat)
    final = _combine_tc(gathered, xf, keptf, pmax)

    counts = counts_l[0, :e]
    psum = psum_l[0, :e]
    n_dropped = jnp.sum(jnp.maximum(counts - capacity, 0.0)).astype(jnp.int32)
    return (final.reshape(b, s, d), counts, psum, n_dropped,
            pmax.reshape(n))


# speedup vs baseline: 5.2882x; 1.0473x over previous
import functools

import jax
import jax.numpy as jnp
from jax import lax
from jax.experimental import pallas as pl
from jax.experimental.pallas import tpu as pltpu
from jax.experimental.pallas import tpu_sc as plsc

_NC = 2
_NS = 16
_NW = _NC * _NS

_LANES = 128
_TB = 2048
_CT = 2560
_CH = 64
_BIG = 2**30


def _router_body(x_ref, w_ref, b_ref,
                 disp_ref, comb_ref, kept_ref, pmax_ref, counts_ref, psum_ref,
                 *, capacity, trash):
    i = pl.program_id(0)

    @pl.when(i == 0)
    def _init():
        counts_ref[...] = jnp.zeros_like(counts_ref)
        psum_ref[...] = jnp.zeros_like(psum_ref)

    x = x_ref[...]
    logits = jnp.dot(x, w_ref[...], preferred_element_type=jnp.float32)
    logits = logits + b_ref[...]
    m = jnp.max(logits, axis=1, keepdims=True)
    ex = jnp.exp(logits - m)
    s = jnp.sum(ex, axis=1, keepdims=True)
    prob = ex / s
    pmax = jnp.max(prob, axis=1, keepdims=True)
    lane = lax.broadcasted_iota(jnp.int32, (_TB, _LANES), 1)
    routes = jnp.min(jnp.where(logits == m, lane, _LANES), axis=1,
                     keepdims=True)
    onehot = (lane == routes).astype(jnp.float32)

    psum_ref[...] += jnp.sum(prob, axis=0, keepdims=True)

    r = lax.broadcasted_iota(jnp.int32, (128, 128), 0)
    c = lax.broadcasted_iota(jnp.int32, (128, 128), 1)
    ltri = (r >= c).astype(jnp.float32)
    run = counts_ref[...]
    parts = []
    for g in range(_TB // 128):
        blk = onehot[g * 128:(g + 1) * 128, :]
        cs = jnp.dot(ltri, blk, preferred_element_type=jnp.float32)
        parts.append(cs + run)
        run = run + cs[127:128, :]
    counts_ref[...] = run
    pos_incl = jnp.concatenate(parts, axis=0)

    posf = jnp.sum(pos_incl * onehot, axis=1, keepdims=True) - 1.0
    pos = posf.astype(jnp.int32)
    keptb = pos < capacity
    clip = jnp.minimum(pos, capacity - 1)
    comb = routes * capacity + clip
    disp_ref[...] = jnp.where(keptb, comb, trash)
    comb_ref[...] = comb
    kept_ref[...] = keptb.astype(jnp.float32)
    pmax_ref[...] = pmax


def _router(xf, w_pad, b_pad, capacity, trash):
    n, d = xf.shape
    nb = n // _TB
    out_shape = [
        jax.ShapeDtypeStruct((n, 1), jnp.int32),
        jax.ShapeDtypeStruct((n, 1), jnp.int32),
        jax.ShapeDtypeStruct((n, 1), jnp.float32),
        jax.ShapeDtypeStruct((n, 1), jnp.float32),
        jax.ShapeDtypeStruct((1, _LANES), jnp.float32),
        jax.ShapeDtypeStruct((1, _LANES), jnp.float32),
    ]
    tok_spec = pl.BlockSpec((_TB, 1), lambda i: (i, 0))
    acc_spec = pl.BlockSpec((1, _LANES), lambda i: (0, 0))
    row_spec = pl.BlockSpec((_TB, d), lambda i: (i, 0))
    return pl.pallas_call(
        functools.partial(_router_body, capacity=capacity, trash=trash),
        grid=(nb,),
        in_specs=[
            row_spec,
            pl.BlockSpec((d, _LANES), lambda i: (0, 0)),
            pl.BlockSpec((1, _LANES), lambda i: (0, 0)),
        ],
        out_specs=[tok_spec, tok_spec, tok_spec, tok_spec, acc_spec, acc_spec],
        out_shape=out_shape,
    )(xf, w_pad, b_pad)


def _sc_dispatch(disp2d, xf, nt):
    n, d = xf.shape
    tpw = n // _NW
    nch = tpw // _CH
    mesh = plsc.VectorSubcoreMesh(core_axis_name="c", subcore_axis_name="s",
                                  num_cores=_NC, num_subcores=_NS)

    dt = xf.dtype

    @functools.partial(
        pl.kernel,
        out_type=jax.ShapeDtypeStruct((nt + 8, d), dt),
        mesh=mesh,
        compiler_params=pltpu.CompilerParams(needs_layout_passes=False),
        scratch_types=[
            pltpu.VMEM((nch, _CH), jnp.int32),
            pltpu.VMEM((2, _CH, d), dt),
            pltpu.SemaphoreType.DMA,
            pltpu.SemaphoreType.DMA,
        ],
    )
    def dispatch(disp_hbm, xf_hbm, buf_hbm, idx_v, rows_v, sem_in, sem_out):
        wid = lax.axis_index("s") * _NC + lax.axis_index("c")
        pltpu.sync_copy(disp_hbm.at[pl.ds(wid * nch, nch)], idx_v)

        def load(k):
            return pltpu.async_copy(
                xf_hbm.at[pl.ds(wid * tpw + k * _CH, _CH)],
                rows_v.at[k % 2], sem_in)

        def scat(k):
            return pltpu.async_copy(
                rows_v.at[k % 2], buf_hbm.at[idx_v.at[k]], sem_out)

        h_out = [None, None]
        h_in = load(0)
        for k in range(nch):
            h_in.wait()
            if k >= 1:
                h_out[(k - 1) % 2].wait()
            if k + 1 < nch:
                h_in = load(k + 1)
            h_out[k % 2] = scat(k)
        h_out[(nch - 1) % 2].wait()

    return dispatch(disp2d, xf)


def _ffn_body(x_ref, w1_ref, b1_ref, w2_ref, b2_ref, o_ref):
    x = x_ref[...]
    h = jnp.dot(x, w1_ref[0], preferred_element_type=jnp.float32)
    h = jnp.maximum(h + b1_ref[0], 0.0)
    o = jnp.dot(h, w2_ref[0], preferred_element_type=jnp.float32)
    o_ref[...] = o + b2_ref[0]


def _ffn(buf_flat, w1, b1, w2, b2, e, cap):
    d = buf_flat.shape[1]
    h = w1.shape[2]
    ct = cap // _CT
    return pl.pallas_call(
        _ffn_body,
        grid=(e, ct),
        in_specs=[
            pl.BlockSpec((_CT, d), lambda i, j: (i * ct + j, 0)),
            pl.BlockSpec((1, d, h), lambda i, j: (i, 0, 0)),
            pl.BlockSpec((1, 1, h), lambda i, j: (i, 0, 0)),
            pl.BlockSpec((1, h, d), lambda i, j: (i, 0, 0)),
            pl.BlockSpec((1, 1, d), lambda i, j: (i, 0, 0)),
        ],
        out_specs=pl.BlockSpec((_CT, d), lambda i, j: (i * ct + j, 0)),
        out_shape=jax.ShapeDtypeStruct((e * cap, d), jnp.float32),
    )(buf_flat, w1, b1.reshape(e, 1, h), w2, b2.reshape(e, 1, d))


def _sc_combine(comb, out_flat):
    n = comb.shape[0]
    d = out_flat.shape[1]
    tpw = n // _NW
    mesh = plsc.VectorSubcoreMesh(core_axis_name="c", subcore_axis_name="s",
                                  num_cores=_NC, num_subcores=_NS)

    dt = out_flat.dtype

    @functools.partial(
        pl.kernel,
        out_type=jax.ShapeDtypeStruct((n, d), dt),
        mesh=mesh,
        compiler_params=pltpu.CompilerParams(needs_layout_passes=False),
        scratch_types=[
            pltpu.VMEM((tpw,), jnp.int32),
            pltpu.VMEM((2, _CH, d), dt),
            pltpu.SemaphoreType.DMA,
            pltpu.SemaphoreType.DMA,
        ],
    )
    def combine(comb_hbm, out_hbm, g_hbm, cidx_v, rows_v, sem_in, sem_out):
        wid = lax.axis_index("s") * _NC + lax.axis_index("c")
        tbase = wid * tpw
        nch = tpw // _CH
        pltpu.sync_copy(comb_hbm.at[pl.ds(tbase, tpw)], cidx_v)

        def load(k):
            return pltpu.async_copy(
                out_hbm.at[cidx_v.at[pl.ds(k * _CH, _CH)]],
                rows_v.at[k % 2], sem_in)

        def store(k):
            return pltpu.async_copy(
                rows_v.at[k % 2], g_hbm.at[pl.ds(tbase + k * _CH, _CH)],
                sem_out)

        h_out = [None, None]
        h_in = load(0)
        for k in range(nch):
            h_in.wait()
            if k >= 1:
                h_out[(k - 1) % 2].wait()
            if k + 1 < nch:
                h_in = load(k + 1)
            h_out[k % 2] = store(k)
        h_out[(nch - 1) % 2].wait()

    return combine(comb, out_flat)


def _combine_body(g_ref, x_ref, k_ref, p_ref, o_ref):
    keep = k_ref[...] > 0.0
    o_ref[...] = jnp.where(keep, g_ref[...], x_ref[...]) * p_ref[...]


def _combine_tc(gathered, xf, keptf, pmax):
    n, d = xf.shape
    row_spec = pl.BlockSpec((_TB, d), lambda i: (i, 0))
    col_spec = pl.BlockSpec((_TB, 1), lambda i: (i, 0))
    return pl.pallas_call(
        _combine_body,
        grid=(n // _TB,),
        in_specs=[row_spec, row_spec, col_spec, col_spec],
        out_specs=row_spec,
        out_shape=jax.ShapeDtypeStruct((n, d), jnp.float32),
    )(gathered, xf, keptf, pmax)


def kernel(x, W_switch, b_switch, W1, b1, W2, b2):
    b, s, d = x.shape
    e = W_switch.shape[1]
    n = b * s
    capacity = int(1.25 * n / e)
    nt = e * capacity
    xf = x.reshape(n, d)

    w_pad = jnp.zeros((d, _LANES), jnp.float32).at[:, :e].set(W_switch)
    b_pad = jnp.full((1, _LANES), -1e30, jnp.float32).at[0, :e].set(b_switch)

    disp, comb, keptf, pmax, counts_l, psum_l = _router(xf, w_pad, b_pad,
                                                        capacity, nt)

    buf = _sc_dispatch(disp.reshape(n // _CH, _CH), xf, nt)
    out_flat = _ffn(buf, W1, b1, W2, b2, e, capacity)
    gathered = _sc_combine(comb.reshape(n), out_flat)
    final = _combine_tc(gathered, xf, keptf, pmax)

    counts = counts_l[0, :e]
    psum = psum_l[0, :e]
    n_dropped = jnp.sum(jnp.maximum(counts - capacity, 0.0)).astype(jnp.int32)
    return (final.reshape(b, s, d), counts, psum, n_dropped,
            pmax.reshape(n))
